# baseline (speedup 1.0000x reference)
import functools
import math

import jax
import jax.numpy as jnp
from jax import lax
from jax.experimental import pallas as pl
from jax.experimental.pallas import tpu as pltpu
from jax.experimental.pallas import tpu_sc as plsc

_D = 32
_SCALE = math.sqrt(float(_D))
_NC, _NS = 2, 16
_NW = _NC * _NS
_C = 1600


@functools.lru_cache(maxsize=None)
def _make(B: int):
    b_per_w = B // _NW
    n_chunks = b_per_w // _C
    assert b_per_w % _C == 0 and B % (8 * _NW) == 0

    mesh = plsc.VectorSubcoreMesh(
        core_axis_name="c", subcore_axis_name="s",
        num_cores=_NC, num_subcores=_NS,
    )

    @functools.partial(
        pl.kernel,
        out_type=jax.ShapeDtypeStruct((B, _D), jnp.float32),
        mesh=mesh,
        scratch_types=[
            pltpu.VMEM((_C,), jnp.int32),
            pltpu.VMEM((_C, _D), jnp.float32),
            pltpu.SemaphoreType.DMA,
        ],
        compiler_params=pltpu.CompilerParams(use_tc_tiling_on_sc=False),
    )
    def k(idx_hbm, table_hbm, out_hbm, idx_v, rows_v, sem):
        wid = lax.axis_index("s") * _NC + lax.axis_index("c")
        wbase = wid * b_per_w

        def chunk(g, carry):
            base = wbase + g * _C
            pltpu.sync_copy(idx_hbm.at[pl.ds(base, _C)], idx_v)
            pltpu.async_copy(table_hbm.at[idx_v], rows_v, sem).wait()

            def scale_row(i, c):
                for j in range(_D // 16):
                    sl = pl.ds(j * 16, 16)
                    rows_v[i, sl] = rows_v[i, sl] * _SCALE
                return c

            lax.fori_loop(0, _C, scale_row, 0)
            pltpu.sync_copy(rows_v, out_hbm.at[pl.ds(base, _C)])
            return carry

        lax.fori_loop(0, n_chunks, chunk, 0)

    return k


def kernel(x, table):
    s0, s1 = x.shape
    B = s0 * s1
    idx = x.reshape(B).astype(jnp.int32)
    out = _make(B)(idx, table)
    return out.reshape(s0, s1, _D)


# speedup vs baseline: 1.0777x; 1.0777x over previous
import functools
import math

import jax
import jax.numpy as jnp
from jax import lax
from jax.experimental import pallas as pl
from jax.experimental.pallas import tpu as pltpu
from jax.experimental.pallas import tpu_sc as plsc

_D = 32
_SCALE = math.sqrt(float(_D))
_NC, _NS = 2, 16
_NW = _NC * _NS
_C = 1280
_RU = 8


@functools.lru_cache(maxsize=None)
def _make(B: int):
    b_per_w = B // _NW
    n_chunks = b_per_w // _C
    assert b_per_w % _C == 0 and B % (8 * _NW) == 0
    assert _C % _RU == 0 and n_chunks % 2 == 0

    mesh = plsc.VectorSubcoreMesh(
        core_axis_name="c", subcore_axis_name="s",
        num_cores=_NC, num_subcores=_NS,
    )

    @functools.partial(
        pl.kernel,
        out_type=jax.ShapeDtypeStruct((B, _D), jnp.float32),
        mesh=mesh,
        scratch_types=[
            pltpu.VMEM((b_per_w,), jnp.int32),
            pltpu.VMEM((2, _C, _D), jnp.float32),
            pltpu.SemaphoreType.DMA,
            pltpu.SemaphoreType.DMA,
            pltpu.SemaphoreType.DMA,
            pltpu.SemaphoreType.DMA,
        ],
        compiler_params=pltpu.CompilerParams(use_tc_tiling_on_sc=False),
    )
    def k(idx_hbm, table_hbm, out_hbm, idx_v, rows_v, g0, g1, w0, w1):
        wid = lax.axis_index("s") * _NC + lax.axis_index("c")
        wbase = wid * b_per_w
        gsems = (g0, g1)
        wsems = (w0, w1)

        pltpu.sync_copy(idx_hbm.at[pl.ds(wbase, b_per_w)], idx_v)

        def gather_start(g, buf):
            pltpu.async_copy(
                table_hbm.at[idx_v.at[pl.ds(g * _C, _C)]],
                rows_v.at[buf], gsems[buf])

        def gather_wait(g, buf):
            pltpu.make_async_copy(
                table_hbm.at[idx_v.at[pl.ds(g * _C, _C)]],
                rows_v.at[buf], gsems[buf]).wait()

        def wb_start(g, buf):
            pltpu.async_copy(
                rows_v.at[buf], out_hbm.at[pl.ds(wbase + g * _C, _C)],
                wsems[buf])

        def wb_wait(g, buf):
            pltpu.make_async_copy(
                rows_v.at[buf], out_hbm.at[pl.ds(wbase + g * _C, _C)],
                wsems[buf]).wait()

        def scale(buf):
            rv = rows_v.at[buf]

            def blk(i, c):
                for r in range(_RU):
                    for j in range(_D // 16):
                        sl = pl.ds(j * 16, 16)
                        rv[i * _RU + r, sl] = rv[i * _RU + r, sl] * _SCALE
                return c

            lax.fori_loop(0, _C // _RU, blk, 0, unroll=False)

        gather_start(0, 0)
        gather_start(1, 1)

        def pair(h, carry):
            for buf in range(2):
                g = 2 * h + buf
                gather_wait(g, buf)
                scale(buf)
                wb_start(g, buf)

                @pl.when(g + 2 < n_chunks)
                def _():
                    wb_wait(g, buf)
                    gather_start(g + 2, buf)
            return carry

        lax.fori_loop(0, n_chunks // 2, pair, 0)

        for buf in range(2):
            wb_wait(n_chunks - 2 + buf, buf)

    return k


def kernel(x, table):
    s0, s1 = x.shape
    B = s0 * s1
    idx = x.reshape(B).astype(jnp.int32)
    out = _make(B)(idx, table)
    return out.reshape(s0, s1, _D)


# speedup vs baseline: 1.4887x; 1.3814x over previous
import functools
import math

import jax
import jax.numpy as jnp
from jax import lax
from jax.experimental import pallas as pl
from jax.experimental.pallas import tpu as pltpu
from jax.experimental.pallas import tpu_sc as plsc

_D = 32
_J = 50
_NI = 16384
_SCALE = math.sqrt(float(_D))
_NC, _NS = 2, 16
_NW = _NC * _NS
_CB = _NI // 128
_CPW = _CB // _NW
_N = _CPW * _J
_NBUF = 3


def _make():
    mesh = plsc.VectorSubcoreMesh(
        core_axis_name="c", subcore_axis_name="s",
        num_cores=_NC, num_subcores=_NS,
    )

    @functools.partial(
        pl.kernel,
        out_type=jax.ShapeDtypeStruct((_J, _D // 8, _CB, 8, 128), jnp.float32),
        mesh=mesh,
        scratch_types=[
            pltpu.VMEM((128,), jnp.int32),
            pltpu.VMEM((128,), jnp.int32),
            pltpu.VMEM((128,), jnp.int32),
            pltpu.VMEM((_NBUF, 128, _D), jnp.float32),
            pltpu.VMEM((_NBUF, _D, 128), jnp.float32),
            pltpu.SemaphoreType.DMA,
            pltpu.SemaphoreType.DMA,
            pltpu.SemaphoreType.DMA,
            pltpu.SemaphoreType.DMA,
            pltpu.SemaphoreType.DMA,
            pltpu.SemaphoreType.DMA,
            pltpu.SemaphoreType.DMA,
            pltpu.SemaphoreType.DMA,
            pltpu.SemaphoreType.DMA,
        ],
        compiler_params=pltpu.CompilerParams(
            use_tc_tiling_on_sc=False, needs_layout_passes=False),
    )
    def k(idxp_hbm, table_hbm, out_hbm, ic0, ic1, ic2, rows_v, obuf_v,
          i0, i1, i2, g0, g1, g2, w0, w1, w2):
        idxcs = (ic0, ic1, ic2)
        wid = lax.axis_index("s") * _NC + lax.axis_index("c")
        isems = (i0, i1, i2)
        gsems = (g0, g1, g2)
        wsems = (w0, w1, w2)

        def unit_cj(u):
            c_loc = u // _J
            j = u - c_loc * _J
            c = wid * _CPW + c_loc
            return c, j

        def idx_slice(u):
            c, j = unit_cj(u)
            return idxp_hbm.at[pl.ds(j * _NI + c * 128, 128)]

        def idx_start(u, b):
            pltpu.async_copy(idx_slice(u), idxcs[b], isems[b])

        def idx_wait(u, b):
            pltpu.make_async_copy(idx_slice(u), idxcs[b], isems[b]).wait()

        def gather_start(b):
            pltpu.async_copy(table_hbm.at[idxcs[b]], rows_v.at[b], gsems[b])

        def gather_wait(b):
            pltpu.make_async_copy(table_hbm.at[idxcs[b]], rows_v.at[b],
                                  gsems[b]).wait()

        lanes = lax.iota(jnp.int32, 16)
        rowvecs = [lanes + (16 * g) for g in range(8)]

        def xpose_scale(b):
            rv = rows_v.at[b]
            ov = obuf_v.at[b]
            for d in range(_D):
                dcol = jnp.full((16,), d, jnp.int32)
                for g in range(8):
                    v = plsc.load_gather(rv, [rowvecs[g], dcol])
                    ov[d, pl.ds(16 * g, 16)] = v * _SCALE

        def scatter_cols(u, b):
            c, j = unit_cj(u)
            for dr in range(_D // 8):
                pltpu.async_copy(
                    obuf_v.at[b, pl.ds(dr * 8, 8), :],
                    out_hbm.at[j, dr, c, :, :],
                    wsems[b])

        def scatter_drain(b):
            pltpu.make_async_copy(table_hbm.at[pl.ds(0, 128)], rows_v.at[b],
                                  wsems[b]).wait()

        for u0 in range(_NBUF):
            idx_start(u0, u0)
        idx_wait(0, 0)
        gather_start(0)
        idx_wait(1, 1)
        gather_start(1)

        def step(u, carry):
            for b in range(_NBUF):
                @pl.when(u % _NBUF == b)
                def _():
                    gather_wait(b)
                    xpose_scale(b)
                    scatter_cols(u, b)

                    @pl.when(u + _NBUF < _N)
                    def _():
                        idx_start(u + _NBUF, b)

                    b2 = (b + 2) % _NBUF

                    @pl.when(u + 2 < _N)
                    def _():
                        @pl.when(u >= 1)
                        def _():
                            scatter_drain(b2)
                        idx_wait(u + 2, b2)
                        gather_start(b2)
            return carry

        lax.fori_loop(0, _N, step, 0)

        scatter_drain((_N - 3) % _NBUF)
        scatter_drain((_N - 2) % _NBUF)
        scatter_drain((_N - 1) % _NBUF)

    return k


_K = None


def kernel(x, table):
    global _K
    if _K is None:
        _K = _make()
    idxp = x.T.reshape(_J * _NI).astype(jnp.int32)
    P = _K(idxp, table)
    return P.transpose(2, 4, 0, 1, 3).reshape(_NI, _J, _D)


# speedup vs baseline: 1.5201x; 1.0211x over previous
import functools
import math

import jax
import jax.numpy as jnp
from jax import lax
from jax.experimental import pallas as pl
from jax.experimental.pallas import tpu as pltpu
from jax.experimental.pallas import tpu_sc as plsc

_D = 32
_J = 50
_NI = 16384
_SCALE = math.sqrt(float(_D))
_NC, _NS = 2, 16
_NW = _NC * _NS
_CB = _NI // 128
_CPW = _CB // _NW
_R = _CPW * 128
_N = _J
_NBUF = 3


def _make():
    mesh = plsc.VectorSubcoreMesh(
        core_axis_name="c", subcore_axis_name="s",
        num_cores=_NC, num_subcores=_NS,
    )

    @functools.partial(
        pl.kernel,
        out_type=jax.ShapeDtypeStruct((_J, _D // 8, _CB, 8, 128), jnp.float32),
        mesh=mesh,
        scratch_types=[
            pltpu.VMEM((_R,), jnp.int32),
            pltpu.VMEM((_R,), jnp.int32),
            pltpu.VMEM((_R,), jnp.int32),
            pltpu.VMEM((_NBUF, _R, _D), jnp.float32),
            pltpu.VMEM((_NBUF, _D // 8, _CPW, 8, 128), jnp.float32),
            pltpu.SemaphoreType.DMA,
            pltpu.SemaphoreType.DMA,
            pltpu.SemaphoreType.DMA,
            pltpu.SemaphoreType.DMA,
            pltpu.SemaphoreType.DMA,
            pltpu.SemaphoreType.DMA,
            pltpu.SemaphoreType.DMA,
            pltpu.SemaphoreType.DMA,
            pltpu.SemaphoreType.DMA,
        ],
        compiler_params=pltpu.CompilerParams(
            use_tc_tiling_on_sc=False, needs_layout_passes=False),
    )
    def k(idxp_hbm, table_hbm, out_hbm, ic0, ic1, ic2, rows_v, obuf_v,
          i0, i1, i2, g0, g1, g2, w0, w1, w2):
        idxcs = (ic0, ic1, ic2)
        wid = lax.axis_index("s") * _NC + lax.axis_index("c")
        isems = (i0, i1, i2)
        gsems = (g0, g1, g2)
        wsems = (w0, w1, w2)
        cbase = wid * _CPW

        def idx_slice(j):
            return idxp_hbm.at[pl.ds(j * _NI + wid * _R, _R)]

        def idx_start(j, b):
            pltpu.async_copy(idx_slice(j), idxcs[b], isems[b])

        def idx_wait(j, b):
            pltpu.make_async_copy(idx_slice(j), idxcs[b], isems[b]).wait()

        def gather_start(b):
            pltpu.async_copy(table_hbm.at[idxcs[b]], rows_v.at[b], gsems[b])

        def gather_wait(b):
            pltpu.make_async_copy(table_hbm.at[idxcs[b]], rows_v.at[b],
                                  gsems[b]).wait()

        lanes = lax.iota(jnp.int32, 16)
        zeros16 = jnp.zeros((16,), jnp.int32)
        rowvecs = [[lanes + (cl * 128 + 16 * g) for g in range(8)]
                   for cl in range(_CPW)]

        def xpose_scale(b):
            rv = rows_v.at[b]
            ov = obuf_v.at[b]

            def body(d, carry):
                dvec = zeros16 + d
                dr = d // 8
                d8 = d - dr * 8
                for cl in range(_CPW):
                    for g in range(8):
                        v = plsc.load_gather(rv, [rowvecs[cl][g], dvec])
                        ov[dr, cl, d8, pl.ds(16 * g, 16)] = v * _SCALE
                return carry

            lax.fori_loop(0, _D, body, 0, unroll=False)

        def out_start(j, b):
            for dr in range(_D // 8):
                pltpu.async_copy(
                    obuf_v.at[b, dr],
                    out_hbm.at[j, dr, pl.ds(cbase, _CPW), :, :],
                    wsems[b])

        def out_drain(b):
            pltpu.make_async_copy(table_hbm.at[pl.ds(0, _R)], rows_v.at[b],
                                  wsems[b]).wait()

        for u0 in range(_NBUF):
            idx_start(u0, u0)
        idx_wait(0, 0)
        gather_start(0)
        idx_wait(1, 1)
        gather_start(1)

        def step(u, carry):
            for b in range(_NBUF):
                @pl.when(u % _NBUF == b)
                def _():
                    gather_wait(b)
                    xpose_scale(b)
                    out_start(u, b)

                    @pl.when(u + _NBUF < _N)
                    def _():
                        idx_start(u + _NBUF, b)

                    b2 = (b + 2) % _NBUF

                    @pl.when(u + 2 < _N)
                    def _():
                        @pl.when(u >= 1)
                        def _():
                            out_drain(b2)
                        idx_wait(u + 2, b2)
                        gather_start(b2)
            return carry

        lax.fori_loop(0, _N, step, 0)

        out_drain((_N - 3) % _NBUF)
        out_drain((_N - 2) % _NBUF)
        out_drain((_N - 1) % _NBUF)

    return k


_K = None


def kernel(x, table):
    global _K
    if _K is None:
        _K = _make()
    idxp = x.T.reshape(_J * _NI).astype(jnp.int32)
    P = _K(idxp, table)
    return P.transpose(2, 4, 0, 1, 3).reshape(_NI, _J, _D)
